# baseline (speedup 1.0000x reference)
import functools

import jax
import jax.numpy as jnp
from jax import lax
from jax.experimental import pallas as pl
from jax.experimental.pallas import tpu as pltpu
from jax.experimental.pallas import tpu_sc as plsc

E = 6_400_000
NSEG = 200
STRENGTH = 0.001

NC = 2
NS = 16
NW = NC * NS
LANES = 16

ACC = 256
ACCW = ACC * LANES
PER_TILE = E // NW
CHUNK = 20_000
NCHUNK = PER_TILE // CHUNK
VECS = CHUNK // LANES


def _sc_partials(x, idx):
    mesh = plsc.VectorSubcoreMesh(core_axis_name="c", subcore_axis_name="s")

    @functools.partial(
        pl.kernel,
        mesh=mesh,
        compiler_params=pltpu.CompilerParams(needs_layout_passes=False),
        out_type=jax.ShapeDtypeStruct((2, NW, ACCW), jnp.float32),
        scratch_types=[
            pltpu.VMEM((CHUNK,), jnp.float32),
            pltpu.VMEM((CHUNK,), jnp.float32),
            pltpu.VMEM((CHUNK,), jnp.int32),
            pltpu.VMEM((CHUNK,), jnp.int32),
            pltpu.VMEM((ACCW,), jnp.float32),
            pltpu.VMEM((ACCW,), jnp.float32),
            pltpu.SemaphoreType.DMA,
            pltpu.SemaphoreType.DMA,
        ],
    )
    def k(x_hbm, idx_hbm, out_hbm, x_buf0, x_buf1, i_buf0, i_buf1,
          acc_s, acc_c, sem0, sem1):
        wid = lax.axis_index("s") * NC + lax.axis_index("c")
        base = wid * PER_TILE
        sems = (sem0, sem1)
        x_bufs = (x_buf0, x_buf1)
        i_bufs = (i_buf0, i_buf1)

        zeros16 = jnp.zeros((LANES,), jnp.float32)

        @plsc.parallel_loop(0, ACCW, step=LANES, unroll=8)
        def _(j):
            acc_s[pl.ds(j, LANES)] = zeros16
            acc_c[pl.ds(j, LANES)] = zeros16

        ones16 = jnp.ones((LANES,), jnp.float32)
        lane16 = lax.broadcasted_iota(jnp.int32, (LANES,), 0)

        def start(c, slot):
            off = base + c * CHUNK
            pltpu.async_copy(
                x_hbm.at[pl.ds(off, CHUNK)], x_bufs[slot], sems[slot])
            pltpu.async_copy(
                idx_hbm.at[pl.ds(off, CHUNK)], i_bufs[slot], sems[slot])

        def wait(slot):
            pltpu.make_async_copy(
                x_hbm.at[pl.ds(0, CHUNK)], x_bufs[slot], sems[slot]).wait()
            pltpu.make_async_copy(
                idx_hbm.at[pl.ds(0, CHUNK)], i_bufs[slot], sems[slot]).wait()

        start(0, 0)
        start(1, 1)

        @pl.loop(0, NCHUNK, step=2)
        def _(c):
            for b in range(2):
                wait(b)

                @plsc.parallel_loop(0, CHUNK, step=LANES, unroll=12)
                def _(i):
                    xv = x_bufs[b][pl.ds(i, LANES)]
                    iv = i_bufs[b][pl.ds(i, LANES)]
                    ivs = iv * LANES + lane16
                    plsc.addupdate_scatter(acc_s, [ivs], xv)
                    plsc.addupdate_scatter(acc_c, [ivs], ones16)

                @pl.when(c + b + 2 < NCHUNK)
                def _():
                    start(c + b + 2, b)

        pltpu.sync_copy(acc_s, out_hbm.at[0, wid])
        pltpu.sync_copy(acc_c, out_hbm.at[1, wid])

    return k(x, idx)


def _loss_tc(partials, target_pad):
    def body(p_ref, t_ref, o_ref):
        sums = jnp.sum(jnp.sum(p_ref[0], axis=2), axis=0).reshape(1, ACC)
        cnts = jnp.sum(jnp.sum(p_ref[1], axis=2), axis=0).reshape(1, ACC)
        mean = sums / jnp.maximum(cnts, 1.0)
        d = mean - t_ref[...]
        col = lax.broadcasted_iota(jnp.int32, (1, ACC), 1)
        sq = jnp.where(col < NSEG, d * d, 0.0)
        loss = jnp.sum(sq) * jnp.float32(STRENGTH / NSEG)
        o_ref[...] = jnp.broadcast_to(loss, (1, 1))

    return pl.pallas_call(
        body,
        out_shape=jax.ShapeDtypeStruct((1, 1), jnp.float32),
    )(partials, target_pad)


def kernel(x, idx, target_mean_weights):
    partials = _sc_partials(x, idx)
    partials = partials.reshape(2, NW, ACC, LANES)
    tgt = jnp.pad(target_mean_weights, (0, ACC - NSEG)).reshape(1, ACC)
    loss = _loss_tc(partials, tgt)
    return loss[0, 0].astype(jnp.float32)


# speedup vs baseline: 1.0070x; 1.0070x over previous
import functools

import jax
import jax.numpy as jnp
from jax import lax
from jax.experimental import pallas as pl
from jax.experimental.pallas import tpu as pltpu
from jax.experimental.pallas import tpu_sc as plsc

E = 6_400_000
NSEG = 200
STRENGTH = 0.001

NC = 2
NS = 16
NW = NC * NS
LANES = 16

ACC = 256
ACCW = ACC * LANES
PER_TILE = E // NW
CHUNK = 20_000
NCHUNK = PER_TILE // CHUNK
VECS = CHUNK // LANES


def _sc_partials(x, idx):
    mesh = plsc.VectorSubcoreMesh(core_axis_name="c", subcore_axis_name="s")

    @functools.partial(
        pl.kernel,
        mesh=mesh,
        compiler_params=pltpu.CompilerParams(needs_layout_passes=False),
        out_type=jax.ShapeDtypeStruct((2, NW, ACCW), jnp.float32),
        scratch_types=[
            pltpu.VMEM((CHUNK,), jnp.float32),
            pltpu.VMEM((CHUNK,), jnp.float32),
            pltpu.VMEM((CHUNK,), jnp.int32),
            pltpu.VMEM((CHUNK,), jnp.int32),
            pltpu.VMEM((ACCW,), jnp.float32),
            pltpu.VMEM((ACCW,), jnp.float32),
            pltpu.SemaphoreType.DMA,
            pltpu.SemaphoreType.DMA,
        ],
    )
    def k(x_hbm, idx_hbm, out_hbm, x_buf0, x_buf1, i_buf0, i_buf1,
          acc_s, acc_c, sem0, sem1):
        wid = lax.axis_index("s") * NC + lax.axis_index("c")
        base = wid * PER_TILE
        sems = (sem0, sem1)
        x_bufs = (x_buf0, x_buf1)
        i_bufs = (i_buf0, i_buf1)

        zeros16 = jnp.zeros((LANES,), jnp.float32)

        @plsc.parallel_loop(0, ACCW, step=LANES, unroll=8)
        def _(j):
            acc_s[pl.ds(j, LANES)] = zeros16
            acc_c[pl.ds(j, LANES)] = zeros16

        ones16 = jnp.ones((LANES,), jnp.float32)
        lane16 = lax.broadcasted_iota(jnp.int32, (LANES,), 0)

        def start(c, slot):
            off = base + c * CHUNK
            pltpu.async_copy(
                x_hbm.at[pl.ds(off, CHUNK)], x_bufs[slot], sems[slot])
            pltpu.async_copy(
                idx_hbm.at[pl.ds(off, CHUNK)], i_bufs[slot], sems[slot])

        def wait(slot):
            pltpu.make_async_copy(
                x_hbm.at[pl.ds(0, CHUNK)], x_bufs[slot], sems[slot]).wait()
            pltpu.make_async_copy(
                idx_hbm.at[pl.ds(0, CHUNK)], i_bufs[slot], sems[slot]).wait()

        start(0, 0)
        start(1, 1)

        @pl.loop(0, NCHUNK, step=2)
        def _(c):
            for b in range(2):
                wait(b)

                @plsc.parallel_loop(0, CHUNK, step=LANES, unroll=8)
                def _(i):
                    xv = x_bufs[b][pl.ds(i, LANES)]
                    iv = i_bufs[b][pl.ds(i, LANES)]
                    ivs = iv * LANES + lane16
                    plsc.addupdate_scatter(acc_s, [ivs], xv)
                    plsc.addupdate_scatter(acc_c, [ivs], ones16)

                @pl.when(c + b + 2 < NCHUNK)
                def _():
                    start(c + b + 2, b)

        pltpu.sync_copy(acc_s, out_hbm.at[0, wid])
        pltpu.sync_copy(acc_c, out_hbm.at[1, wid])

    return k(x, idx)


def _loss_tc(partials, target_pad):
    def body(p_ref, t_ref, o_ref):
        sums = jnp.sum(jnp.sum(p_ref[0], axis=2), axis=0).reshape(1, ACC)
        cnts = jnp.sum(jnp.sum(p_ref[1], axis=2), axis=0).reshape(1, ACC)
        mean = sums / jnp.maximum(cnts, 1.0)
        tpad = jnp.concatenate(
            [t_ref[...], jnp.zeros((ACC - NSEG,), jnp.float32)]).reshape(1, ACC)
        d = mean - tpad
        col = lax.broadcasted_iota(jnp.int32, (1, ACC), 1)
        sq = jnp.where(col < NSEG, d * d, 0.0)
        loss = jnp.sum(sq) * jnp.float32(STRENGTH / NSEG)
        o_ref[...] = jnp.broadcast_to(loss, (1, 1))

    return pl.pallas_call(
        body,
        out_shape=jax.ShapeDtypeStruct((1, 1), jnp.float32),
    )(partials, target_pad)


def kernel(x, idx, target_mean_weights):
    partials = _sc_partials(x, idx)
    partials = partials.reshape(2, NW, ACC, LANES)
    loss = _loss_tc(partials, target_mean_weights)
    return loss[0, 0].astype(jnp.float32)
